# baseline (speedup 1.0000x reference)
import functools

import jax
import jax.numpy as jnp
from jax import lax
from jax.experimental import pallas as pl
from jax.experimental.pallas import tpu as pltpu
from jax.experimental.pallas import tpu_sc as plsc


def _make_kernel(B, S, D):
    mesh = plsc.ScalarSubcoreMesh(axis_name="c", num_cores=2)
    per_core = B // 2

    @functools.partial(
        pl.kernel,
        out_type=jax.ShapeDtypeStruct((B, D), jnp.float32),
        mesh=mesh,
        scratch_types=[
            pltpu.SMEM((B,), jnp.int32),
            pltpu.SemaphoreType.DMA,
        ],
    )
    def last_pool(x_hbm, leng_hbm, out_hbm, leng_s, sem):
        cid = lax.axis_index("c")
        base = cid * per_core
        pltpu.sync_copy(leng_hbm, leng_s)
        copies = []
        for j in range(per_core):
            b = base + j
            idx = lax.rem(leng_s[b] + (S - 1), S)
            copies.append(
                pltpu.make_async_copy(x_hbm.at[b, idx], out_hbm.at[b], sem)
            )
        for c in copies:
            c.start()
        for c in copies:
            c.wait()

    return last_pool


def kernel(x, leng):
    B, S, D = x.shape
    return _make_kernel(B, S, D)(x, leng.astype(jnp.int32))


# speedup vs baseline: 1.0854x; 1.0854x over previous
import functools

import jax
import jax.numpy as jnp
from jax import lax
from jax.experimental import pallas as pl
from jax.experimental.pallas import tpu as pltpu
from jax.experimental.pallas import tpu_sc as plsc


def _make_kernel(B, S, D):
    mesh = plsc.ScalarSubcoreMesh(axis_name="c", num_cores=1)

    @functools.partial(
        pl.kernel,
        out_type=jax.ShapeDtypeStruct((B, D), jnp.float32),
        mesh=mesh,
        scratch_types=[
            pltpu.SMEM((B,), jnp.int32),
            pltpu.SemaphoreType.DMA,
        ],
    )
    def last_pool(x_hbm, leng_hbm, out_hbm, leng_s, sem):
        pltpu.sync_copy(leng_hbm, leng_s)
        copies = []
        for b in range(B):
            idx = lax.rem(leng_s[b] + (S - 1), S)
            copies.append(
                pltpu.make_async_copy(x_hbm.at[b, idx], out_hbm.at[b], sem)
            )
        for c in copies:
            c.start()
        for c in copies:
            c.wait()

    return last_pool


def kernel(x, leng):
    B, S, D = x.shape
    return _make_kernel(B, S, D)(x, leng.astype(jnp.int32))
